# Initial kernel scaffold; baseline (speedup 1.0000x reference)
import functools

import jax
import jax.numpy as jnp
from jax import lax
from jax.experimental import pallas as pl
from jax.experimental.pallas import tpu as pltpu
from jax.experimental.pallas import tpu_sc as plsc

NC = 2
NS = 16
NT = NC * NS
CH = 128
CW = 16
D = 128


def _make_sc_agg(n_pad, k_chunks, with_cnt):
  mesh = plsc.VectorSubcoreMesh(core_axis_name="c", subcore_axis_name="s")
  rows_per_tile = n_pad // NS
  n_zero = rows_per_tile // 16

  out_type = [jax.ShapeDtypeStruct((NC, n_pad, D), jnp.float32)]
  scratch = [
      pltpu.VMEM((k_chunks, CH), jnp.int32),
      pltpu.VMEM((k_chunks, CH), jnp.int32),
      pltpu.VMEM((CH, D), jnp.float32),
      pltpu.VMEM((16, D), jnp.float32),
      pltpu.VMEM_SHARED((n_pad, D), jnp.float32),
      pltpu.SemaphoreType.DMA,
  ]
  if with_cnt:
    out_type.append(jax.ShapeDtypeStruct((NC, n_pad, CW), jnp.float32))
    scratch += [
        pltpu.VMEM((CH, CW), jnp.float32),
        pltpu.VMEM((16, CW), jnp.float32),
        pltpu.VMEM_SHARED((n_pad, CW), jnp.float32),
    ]

  def body(feat_hbm, src_hbm, dst_hbm, agg_out, *rest):
    if with_cnt:
      (cnt_out, src_v, dst_v, rows_v, zrow_v, agg_sh, sem,
       ones_v, zcnt_v, cnt_sh) = rest
    else:
      src_v, dst_v, rows_v, zrow_v, agg_sh, sem = rest
    cid = lax.axis_index("c")
    sid = lax.axis_index("s")
    wid = cid * NS + sid

    pltpu.sync_copy(src_hbm.at[pl.ds(wid * k_chunks, k_chunks)], src_v)
    pltpu.sync_copy(dst_hbm.at[pl.ds(wid * k_chunks, k_chunks)], dst_v)

    zero16 = jnp.zeros((16,), jnp.float32)
    for i in range(16):
      for j in range(D // 16):
        zrow_v[i, pl.ds(j * 16, 16)] = zero16
    if with_cnt:
      one16 = jnp.ones((16,), jnp.float32)
      for i in range(CH):
        ones_v[i, :] = one16
      for i in range(16):
        zcnt_v[i, :] = zero16

    def zero_body(i, c):
      base = sid * rows_per_tile + i * 16
      pltpu.sync_copy(zrow_v, agg_sh.at[pl.ds(base, 16)])
      if with_cnt:
        pltpu.sync_copy(zcnt_v, cnt_sh.at[pl.ds(base, 16)])
      return c
    lax.fori_loop(0, n_zero, zero_body, 0)
    plsc.subcore_barrier()

    def edge_body(j, c):
      pltpu.async_copy(feat_hbm.at[src_v.at[j]], rows_v, sem).wait()
      pltpu.sync_copy(rows_v, agg_sh.at[dst_v.at[j]], add=True)
      if with_cnt:
        pltpu.sync_copy(ones_v, cnt_sh.at[dst_v.at[j]], add=True)
      return c
    lax.fori_loop(0, k_chunks, edge_body, 0)
    plsc.subcore_barrier()

    base = sid * rows_per_tile
    pltpu.sync_copy(agg_sh.at[pl.ds(base, rows_per_tile)],
                    agg_out.at[cid, pl.ds(base, rows_per_tile)])
    if with_cnt:
      pltpu.sync_copy(cnt_sh.at[pl.ds(base, rows_per_tile)],
                      cnt_out.at[cid, pl.ds(base, rows_per_tile)])

  return functools.partial(
      pl.kernel, out_type=out_type, mesh=mesh, scratch_types=scratch)(body)


def _tc_fuse(a0, a1, c0, c1, feat, w_l, b, w_r, relu):
  n = feat.shape[0]
  blk = 400
  grid = n // blk

  def body(a0_ref, a1_ref, c0_ref, c1_ref, f_ref, wl_ref, b_ref, wr_ref,
           o_ref):
    cnt = c0_ref[:, 0:1] + c1_ref[:, 0:1]
    mean = (a0_ref[...] + a1_ref[...]) / jnp.maximum(cnt, 1.0)
    dn = (((1,), (1,)), ((), ()))
    acc = lax.dot_general(mean, wl_ref[...], dn,
                          preferred_element_type=jnp.float32)
    acc += lax.dot_general(f_ref[...], wr_ref[...], dn,
                           preferred_element_type=jnp.float32)
    acc += b_ref[...]
    if relu:
      acc = jnp.maximum(acc, 0.0)
    o_ref[...] = acc

  row_spec = pl.BlockSpec((blk, D), lambda i: (i, 0))
  cnt_spec = pl.BlockSpec((blk, CW), lambda i: (i, 0))
  full_spec = pl.BlockSpec((D, D), lambda i: (0, 0))
  b_spec = pl.BlockSpec((1, D), lambda i: (0, 0))
  return pl.pallas_call(
      body,
      grid=(grid,),
      in_specs=[row_spec, row_spec, cnt_spec, cnt_spec, row_spec,
                full_spec, b_spec, full_spec],
      out_specs=row_spec,
      out_shape=jax.ShapeDtypeStruct((n, D), jnp.float32),
  )(a0, a1, c0, c1, feat, w_l, b, w_r)


def kernel(x, edge_index, W1_l, b1, W1_r, W2_l, b2, W2_r):
  n = x.shape[0]
  e = edge_index.shape[1]
  n_pad = ((n // 256) + 1) * 256
  k_chunks = -(-e // (NT * CH))
  e_pad = NT * CH * k_chunks

  src = edge_index[0]
  dst = edge_index[1]
  pad = e_pad - e
  src_p = jnp.concatenate([src, jnp.zeros((pad,), jnp.int32)])
  dst_p = jnp.concatenate([dst, jnp.full((pad,), n, jnp.int32)])
  src2d = src_p.reshape(NT * k_chunks, CH)
  dst2d = dst_p.reshape(NT * k_chunks, CH)

  sc_agg_cnt = _make_sc_agg(n_pad, k_chunks, with_cnt=True)
  sc_agg = _make_sc_agg(n_pad, k_chunks, with_cnt=False)

  agg1, cnt = sc_agg_cnt(x, src2d, dst2d)
  b1r = b1.reshape(1, D)
  b2r = b2.reshape(1, D)
  h = _tc_fuse(agg1[0, :n], agg1[1, :n], cnt[0, :n], cnt[1, :n],
               x, W1_l, b1r, W1_r, relu=True)
  (agg2,) = sc_agg(h, src2d, dst2d)
  out = _tc_fuse(agg2[0, :n], agg2[1, :n], cnt[0, :n], cnt[1, :n],
                 h, W2_l, b2r, W2_r, relu=False)
  return out


# speedup vs baseline: 5.6699x; 5.6699x over previous
import functools

import jax
import jax.numpy as jnp
from jax import lax
from jax.experimental import pallas as pl
from jax.experimental.pallas import tpu as pltpu
from jax.experimental.pallas import tpu_sc as plsc

NC = 2
NS = 16
NT = NC * NS
CH = 64
CW = 128
D = 128


def _make_sc_agg(n_pad, k_chunks):
  mesh = plsc.VectorSubcoreMesh(core_axis_name="c", subcore_axis_name="s")
  rows_per_tile = n_pad // NS
  kb = 8

  @functools.partial(
      pl.kernel,
      out_type=jax.ShapeDtypeStruct((NC, n_pad, D), jnp.float32),
      mesh=mesh,
      scratch_types=[
          pltpu.VMEM((kb, CH), jnp.int32),
          pltpu.VMEM((kb, CH), jnp.int32),
          pltpu.VMEM((CH, D), jnp.float32),
          pltpu.VMEM_SHARED((n_pad, D), jnp.float32),
          pltpu.SemaphoreType.DMA,
      ])
  def body(feat_hbm, src_hbm, dst_hbm, agg_out, src_v, dst_v, rows_v,
           agg_sh, sem):
    cid = lax.axis_index("c")
    sid = lax.axis_index("s")
    wid = cid * NS + sid

    zero16 = jnp.zeros((16,), jnp.float32)

    def fill_body(i, c):
      for j in range(D // 16):
        rows_v[i, pl.ds(j * 16, 16)] = zero16
      return c
    lax.fori_loop(0, CH, fill_body, 0)

    def zero_body(i, c):
      pltpu.sync_copy(rows_v, agg_sh.at[pl.ds(sid * rows_per_tile + i * CH,
                                              CH)])
      return c
    lax.fori_loop(0, rows_per_tile // CH, zero_body, 0)
    plsc.subcore_barrier()

    def block_body(b, c):
      base = wid * k_chunks + b * kb
      pltpu.sync_copy(src_hbm.at[pl.ds(base, kb)], src_v)
      pltpu.sync_copy(dst_hbm.at[pl.ds(base, kb)], dst_v)

      def edge_body(j, c2):
        pltpu.async_copy(feat_hbm.at[src_v.at[j]], rows_v, sem).wait()
        pltpu.sync_copy(rows_v, agg_sh.at[dst_v.at[j]], add=True)
        return c2
      return lax.fori_loop(0, kb, edge_body, c)
    lax.fori_loop(0, k_chunks // kb, block_body, 0)
    plsc.subcore_barrier()

    def wb_body(i, c):
      base = sid * rows_per_tile + i * CH
      pltpu.sync_copy(agg_sh.at[pl.ds(base, CH)], rows_v)
      pltpu.sync_copy(rows_v, agg_out.at[cid, pl.ds(base, CH)])
      return c
    lax.fori_loop(0, rows_per_tile // CH, wb_body, 0)

  return body


def _make_sc_cnt(n_pad, k_chunks):
  mesh = plsc.VectorSubcoreMesh(core_axis_name="c", subcore_axis_name="s")
  rows_per_tile = n_pad // NS
  kb = 8

  @functools.partial(
      pl.kernel,
      out_type=jax.ShapeDtypeStruct((NC, n_pad, CW), jnp.float32),
      mesh=mesh,
      scratch_types=[
          pltpu.VMEM((kb, CH), jnp.int32),
          pltpu.VMEM((CH, CW), jnp.float32),
          pltpu.VMEM_SHARED((n_pad, CW), jnp.float32),
      ])
  def body(dst_hbm, cnt_out, dst_v, ones_v, cnt_sh):
    cid = lax.axis_index("c")
    sid = lax.axis_index("s")
    wid = cid * NS + sid

    zero16 = jnp.zeros((16,), jnp.float32)

    def fill0(i, c):
      for j in range(CW // 16):
        ones_v[i, pl.ds(j * 16, 16)] = zero16
      return c
    lax.fori_loop(0, CH, fill0, 0)

    def zero_body(i, c):
      pltpu.sync_copy(ones_v, cnt_sh.at[pl.ds(sid * rows_per_tile + i * CH,
                                              CH)])
      return c
    lax.fori_loop(0, rows_per_tile // CH, zero_body, 0)

    def fill1(i, c):
      for j in range(CW // 16):
        ones_v[i, pl.ds(j * 16, 16)] = zero16 + 1.0
      return c
    lax.fori_loop(0, CH, fill1, 0)
    plsc.subcore_barrier()

    def block_body(b, c):
      base = wid * k_chunks + b * kb
      pltpu.sync_copy(dst_hbm.at[pl.ds(base, kb)], dst_v)

      def edge_body(j, c2):
        pltpu.sync_copy(ones_v, cnt_sh.at[dst_v.at[j]], add=True)
        return c2
      return lax.fori_loop(0, kb, edge_body, c)
    lax.fori_loop(0, k_chunks // kb, block_body, 0)
    plsc.subcore_barrier()

    def wb_body(i, c):
      base = sid * rows_per_tile + i * CH
      pltpu.sync_copy(cnt_sh.at[pl.ds(base, CH)], ones_v)
      pltpu.sync_copy(ones_v, cnt_out.at[cid, pl.ds(base, CH)])
      return c
    lax.fori_loop(0, rows_per_tile // CH, wb_body, 0)

  return body


def _tc_fuse(a0, a1, c0, c1, feat, w_l, b, w_r, relu):
  n = feat.shape[0]
  blk = 400
  grid = n // blk

  def body(a0_ref, a1_ref, c0_ref, c1_ref, f_ref, wl_ref, b_ref, wr_ref,
           o_ref):
    cnt = c0_ref[:, 0:1] + c1_ref[:, 0:1]
    mean = (a0_ref[...] + a1_ref[...]) / jnp.maximum(cnt, 1.0)
    dn = (((1,), (1,)), ((), ()))
    acc = lax.dot_general(mean, wl_ref[...], dn,
                          preferred_element_type=jnp.float32)
    acc += lax.dot_general(f_ref[...], wr_ref[...], dn,
                           preferred_element_type=jnp.float32)
    acc += b_ref[...]
    if relu:
      acc = jnp.maximum(acc, 0.0)
    o_ref[...] = acc

  row_spec = pl.BlockSpec((blk, D), lambda i: (i, 0))
  cnt_spec = pl.BlockSpec((blk, CW), lambda i: (i, 0))
  full_spec = pl.BlockSpec((D, D), lambda i: (0, 0))
  b_spec = pl.BlockSpec((1, D), lambda i: (0, 0))
  return pl.pallas_call(
      body,
      grid=(grid,),
      in_specs=[row_spec, row_spec, cnt_spec, cnt_spec, row_spec,
                full_spec, b_spec, full_spec],
      out_specs=row_spec,
      out_shape=jax.ShapeDtypeStruct((n, D), jnp.float32),
  )(a0, a1, c0, c1, feat, w_l, b, w_r)


def kernel(x, edge_index, W1_l, b1, W1_r, W2_l, b2, W2_r):
  n = x.shape[0]
  e = edge_index.shape[1]
  n_pad = ((n // 256) + 1) * 256
  k_chunks = -(-e // (NT * CH * 8)) * 8
  e_pad = NT * CH * k_chunks

  src = edge_index[0]
  dst = edge_index[1]
  pad = e_pad - e
  pad_i = jnp.arange(pad, dtype=jnp.int32)
  src_p = jnp.concatenate([src, pad_i % n])
  dst_p = jnp.concatenate([dst, n + pad_i % (n_pad - n)])
  src2d = src_p.reshape(NT * k_chunks, CH)
  dst2d = dst_p.reshape(NT * k_chunks, CH)

  sc_agg = _make_sc_agg(n_pad, k_chunks)
  sc_cnt = _make_sc_cnt(n_pad, k_chunks)

  cnt = sc_cnt(dst2d)
  agg1 = sc_agg(x, src2d, dst2d)
  b1r = b1.reshape(1, D)
  b2r = b2.reshape(1, D)
  h = _tc_fuse(agg1[0, :n], agg1[1, :n], cnt[0, :n], cnt[1, :n],
               x, W1_l, b1r, W1_r, relu=True)
  agg2 = sc_agg(h, src2d, dst2d)
  out = _tc_fuse(agg2[0, :n], agg2[1, :n], cnt[0, :n], cnt[1, :n],
                 h, W2_l, b2r, W2_r, relu=False)
  return out


# speedup vs baseline: 6.9985x; 1.2343x over previous
import functools

import jax
import jax.numpy as jnp
from jax import lax
from jax.experimental import pallas as pl
from jax.experimental.pallas import tpu as pltpu
from jax.experimental.pallas import tpu_sc as plsc

NC = 2
NS = 16
NT = NC * NS
CH = 64
CW = 128
D = 128


def _make_sc_agg(n_pad, k_chunks):
  mesh = plsc.VectorSubcoreMesh(core_axis_name="c", subcore_axis_name="s")
  rows_per_tile = n_pad // NS
  kb = 16

  @functools.partial(
      pl.kernel,
      out_type=jax.ShapeDtypeStruct((NC, n_pad, D), jnp.float32),
      mesh=mesh,
      scratch_types=[
          pltpu.VMEM((kb, CH), jnp.int32),
          pltpu.VMEM((kb, CH), jnp.int32),
          pltpu.VMEM((CH, D), jnp.float32),
          pltpu.VMEM((CH, D), jnp.float32),
          pltpu.VMEM_SHARED((n_pad, D), jnp.float32),
          pltpu.SemaphoreType.DMA,
          pltpu.SemaphoreType.DMA,
      ])
  def body(feat_hbm, src_hbm, dst_hbm, agg_out, src_v, dst_v, rows_a,
           rows_b, agg_sh, sem_g, sem_s):
    cid = lax.axis_index("c")
    sid = lax.axis_index("s")
    wid = cid * NS + sid
    bufs = (rows_a, rows_b)

    zero16 = jnp.zeros((16,), jnp.float32)

    def fill_body(i, c):
      for j in range(D // 16):
        rows_a[i, pl.ds(j * 16, 16)] = zero16
      return c
    lax.fori_loop(0, CH, fill_body, 0)

    def zero_body(i, c):
      pltpu.sync_copy(rows_a, agg_sh.at[pl.ds(sid * rows_per_tile + i * CH,
                                              CH)])
      return c
    lax.fori_loop(0, rows_per_tile // CH, zero_body, 0)
    plsc.subcore_barrier()

    def block_body(b, c):
      base = wid * k_chunks + b * kb
      pltpu.sync_copy(src_hbm.at[pl.ds(base, kb)], src_v)
      pltpu.sync_copy(dst_hbm.at[pl.ds(base, kb)], dst_v)

      pltpu.async_copy(feat_hbm.at[src_v.at[0]], bufs[0], sem_g).wait()
      for j in range(kb):
        p = j % 2
        if j >= 1:
          pltpu.make_async_copy(bufs[1 - p], agg_sh.at[dst_v.at[j]],
                                sem_s).wait()
        if j + 1 < kb:
          g = pltpu.async_copy(feat_hbm.at[src_v.at[j + 1]], bufs[1 - p],
                               sem_g)
        pltpu.async_copy(bufs[p], agg_sh.at[dst_v.at[j]], sem_s, add=True)
        if j + 1 < kb:
          g.wait()
      pltpu.make_async_copy(bufs[(kb - 1) % 2], agg_sh.at[dst_v.at[0]],
                            sem_s).wait()
      return c
    lax.fori_loop(0, k_chunks // kb, block_body, 0)
    plsc.subcore_barrier()

    def wb_body(i, c):
      base = sid * rows_per_tile + i * CH
      pltpu.sync_copy(agg_sh.at[pl.ds(base, CH)], rows_a)
      pltpu.sync_copy(rows_a, agg_out.at[cid, pl.ds(base, CH)])
      return c
    lax.fori_loop(0, rows_per_tile // CH, wb_body, 0)

  return body


def _make_sc_cnt(n_pad, k_chunks, ch):
  mesh = plsc.VectorSubcoreMesh(core_axis_name="c", subcore_axis_name="s")
  rows_per_tile = n_pad // NS
  kb = 8

  @functools.partial(
      pl.kernel,
      out_type=jax.ShapeDtypeStruct((NC, n_pad, CW), jnp.float32),
      mesh=mesh,
      scratch_types=[
          pltpu.VMEM((kb, ch), jnp.int32),
          pltpu.VMEM((ch, CW), jnp.float32),
          pltpu.VMEM_SHARED((n_pad, CW), jnp.float32),
          pltpu.SemaphoreType.DMA,
      ])
  def body(dst_hbm, cnt_out, dst_v, ones_v, cnt_sh, sem):
    cid = lax.axis_index("c")
    sid = lax.axis_index("s")
    wid = cid * NS + sid

    zero16 = jnp.zeros((16,), jnp.float32)

    def fill0(i, c):
      for j in range(CW // 16):
        ones_v[i, pl.ds(j * 16, 16)] = zero16
      return c
    lax.fori_loop(0, ch, fill0, 0)

    nz = rows_per_tile // ch

    def zero_body(i, c):
      pltpu.sync_copy(ones_v, cnt_sh.at[pl.ds(sid * rows_per_tile + i * ch,
                                              ch)])
      return c
    lax.fori_loop(0, nz, zero_body, 0)

    def fill1(i, c):
      for j in range(CW // 16):
        ones_v[i, pl.ds(j * 16, 16)] = zero16 + 1.0
      return c
    lax.fori_loop(0, ch, fill1, 0)
    plsc.subcore_barrier()

    def block_body(b, c):
      base = wid * k_chunks + b * kb
      pltpu.sync_copy(dst_hbm.at[pl.ds(base, kb)], dst_v)
      for j in range(kb):
        pltpu.async_copy(ones_v, cnt_sh.at[dst_v.at[j]], sem, add=True)
      for j in range(kb):
        pltpu.make_async_copy(ones_v, cnt_sh.at[dst_v.at[0]], sem).wait()
      return c
    lax.fori_loop(0, k_chunks // kb, block_body, 0)
    plsc.subcore_barrier()

    def wb_body(i, c):
      base = sid * rows_per_tile + i * ch
      pltpu.sync_copy(cnt_sh.at[pl.ds(base, ch)], ones_v)
      pltpu.sync_copy(ones_v, cnt_out.at[cid, pl.ds(base, ch)])
      return c
    lax.fori_loop(0, nz, wb_body, 0)

  return body


def _tc_fuse(a0, a1, c0, c1, feat, w_l, b, w_r, relu):
  n = feat.shape[0]
  blk = 400
  grid = n // blk

  def body(a0_ref, a1_ref, c0_ref, c1_ref, f_ref, wl_ref, b_ref, wr_ref,
           o_ref):
    cnt = c0_ref[:, 0:1] + c1_ref[:, 0:1]
    mean = (a0_ref[...] + a1_ref[...]) / jnp.maximum(cnt, 1.0)
    dn = (((1,), (1,)), ((), ()))
    acc = lax.dot_general(mean, wl_ref[...], dn,
                          preferred_element_type=jnp.float32)
    acc += lax.dot_general(f_ref[...], wr_ref[...], dn,
                           preferred_element_type=jnp.float32)
    acc += b_ref[...]
    if relu:
      acc = jnp.maximum(acc, 0.0)
    o_ref[...] = acc

  row_spec = pl.BlockSpec((blk, D), lambda i: (i, 0))
  cnt_spec = pl.BlockSpec((blk, CW), lambda i: (i, 0))
  full_spec = pl.BlockSpec((D, D), lambda i: (0, 0))
  b_spec = pl.BlockSpec((1, D), lambda i: (0, 0))
  return pl.pallas_call(
      body,
      grid=(grid,),
      in_specs=[row_spec, row_spec, cnt_spec, cnt_spec, row_spec,
                full_spec, b_spec, full_spec],
      out_specs=row_spec,
      out_shape=jax.ShapeDtypeStruct((n, D), jnp.float32),
  )(a0, a1, c0, c1, feat, w_l, b, w_r)


def kernel(x, edge_index, W1_l, b1, W1_r, W2_l, b2, W2_r):
  n = x.shape[0]
  e = edge_index.shape[1]
  n_pad = ((n // 256) + 1) * 256
  k_chunks = -(-e // (NT * CH * 8)) * 8
  e_pad = NT * CH * k_chunks

  src = edge_index[0]
  dst = edge_index[1]
  pad = e_pad - e
  pad_i = jnp.arange(pad, dtype=jnp.int32)
  src_p = jnp.concatenate([src, pad_i % n])
  dst_p = jnp.concatenate([dst, n + pad_i % (n_pad - n)])
  src2d = src_p.reshape(NT * k_chunks, CH)
  dst2d = dst_p.reshape(NT * k_chunks, CH)
  ch_cnt = 128
  kc = e_pad // (NT * ch_cnt)
  dst2d_cnt = dst_p.reshape(NT * kc, ch_cnt)

  sc_agg = _make_sc_agg(n_pad, k_chunks)
  sc_cnt = _make_sc_cnt(n_pad, kc, ch_cnt)

  cnt = sc_cnt(dst2d_cnt)
  agg1 = sc_agg(x, src2d, dst2d)
  b1r = b1.reshape(1, D)
  b2r = b2.reshape(1, D)
  h = _tc_fuse(agg1[0, :n], agg1[1, :n], cnt[0, :n], cnt[1, :n],
               x, W1_l, b1r, W1_r, relu=True)
  agg2 = sc_agg(h, src2d, dst2d)
  out = _tc_fuse(agg2[0, :n], agg2[1, :n], cnt[0, :n], cnt[1, :n],
                 h, W2_l, b2r, W2_r, relu=False)
  return out


# speedup vs baseline: 7.5606x; 1.0803x over previous
import functools

import jax
import jax.numpy as jnp
from jax import lax
from jax.experimental import pallas as pl
from jax.experimental.pallas import tpu as pltpu
from jax.experimental.pallas import tpu_sc as plsc

NC = 2
NS = 16
NT = NC * NS
CH = 80
D = 128


def _make_sc_agg(n_pad, k_chunks, with_cnt):
  mesh = plsc.VectorSubcoreMesh(core_axis_name="c", subcore_axis_name="s")
  rows_per_tile = n_pad // NS
  kb = 16

  out_type = [jax.ShapeDtypeStruct((NC, n_pad, D), jnp.float32)]
  if with_cnt:
    out_type.append(jax.ShapeDtypeStruct((NC, n_pad, D), jnp.float32))

  @functools.partial(
      pl.kernel,
      out_type=out_type,
      mesh=mesh,
      scratch_types=[
          pltpu.VMEM((kb, CH), jnp.int32),
          pltpu.VMEM((kb, CH), jnp.int32),
          pltpu.VMEM((CH, D), jnp.float32),
          pltpu.VMEM((CH, D), jnp.float32),
          pltpu.VMEM_SHARED((n_pad, D), jnp.float32),
          pltpu.SemaphoreType.DMA,
          pltpu.SemaphoreType.DMA,
      ])
  def body(feat_hbm, src_hbm, dst_hbm, agg_out, *rest):
    if with_cnt:
      cnt_out, src_v, dst_v, rows_a, rows_b, agg_sh, sem_g, sem_s = rest
    else:
      src_v, dst_v, rows_a, rows_b, agg_sh, sem_g, sem_s = rest
    cid = lax.axis_index("c")
    sid = lax.axis_index("s")
    wid = cid * NS + sid
    bufs = (rows_a, rows_b)
    zero16 = jnp.zeros((16,), jnp.float32)

    def fill_a(val):
      def fb(i, c):
        for j in range(D // 16):
          rows_a[i, pl.ds(j * 16, 16)] = zero16 + val
        return c
      lax.fori_loop(0, CH, fb, 0)

    def zero_acc():
      def zb(i, c):
        pltpu.sync_copy(rows_a.at[pl.ds(0, 64)],
                        agg_sh.at[pl.ds(sid * rows_per_tile + i * 64, 64)])
        return c
      lax.fori_loop(0, rows_per_tile // 64, zb, 0)

    def writeback(out_ref):
      def wb(i, c):
        base = sid * rows_per_tile + i * 64
        pltpu.sync_copy(agg_sh.at[pl.ds(base, 64)], rows_b.at[pl.ds(0, 64)])
        pltpu.sync_copy(rows_b.at[pl.ds(0, 64)],
                        out_ref.at[cid, pl.ds(base, 64)])
        return c
      lax.fori_loop(0, rows_per_tile // 64, wb, 0)

    if with_cnt:
      fill_a(0.0)
      zero_acc()
      fill_a(1.0)
      plsc.subcore_barrier()

      def cnt_block(b, c):
        base = wid * k_chunks + b * kb
        pltpu.sync_copy(dst_hbm.at[pl.ds(base, kb)], dst_v)
        for j in range(kb):
          pltpu.async_copy(rows_a, agg_sh.at[dst_v.at[j]], sem_s, add=True)
        for j in range(kb):
          pltpu.make_async_copy(rows_a, agg_sh.at[dst_v.at[0]],
                                sem_s).wait()
        return c
      lax.fori_loop(0, k_chunks // kb, cnt_block, 0)
      plsc.subcore_barrier()
      writeback(cnt_out)
      plsc.subcore_barrier()

    fill_a(0.0)
    zero_acc()
    plsc.subcore_barrier()

    def agg_block(b, c):
      base = wid * k_chunks + b * kb
      pltpu.sync_copy(src_hbm.at[pl.ds(base, kb)], src_v)
      pltpu.sync_copy(dst_hbm.at[pl.ds(base, kb)], dst_v)

      pltpu.async_copy(feat_hbm.at[src_v.at[0]], bufs[0], sem_g).wait()
      for j in range(kb):
        p = j % 2
        if j >= 1:
          pltpu.make_async_copy(bufs[1 - p], agg_sh.at[dst_v.at[j]],
                                sem_s).wait()
        if j + 1 < kb:
          g = pltpu.async_copy(feat_hbm.at[src_v.at[j + 1]], bufs[1 - p],
                               sem_g)
        pltpu.async_copy(bufs[p], agg_sh.at[dst_v.at[j]], sem_s, add=True)
        if j + 1 < kb:
          g.wait()
      pltpu.make_async_copy(bufs[(kb - 1) % 2], agg_sh.at[dst_v.at[0]],
                            sem_s).wait()
      return c
    lax.fori_loop(0, k_chunks // kb, agg_block, 0)
    plsc.subcore_barrier()
    writeback(agg_out)

  return body


def _tc_fuse(a0, a1, c0, c1, feat, w_l, b, w_r, relu):
  n = feat.shape[0]
  blk = 400
  grid = n // blk

  def body(a0_ref, a1_ref, c0_ref, c1_ref, f_ref, wl_ref, b_ref, wr_ref,
           o_ref):
    cnt = c0_ref[:, 0:1] + c1_ref[:, 0:1]
    mean = (a0_ref[...] + a1_ref[...]) / jnp.maximum(cnt, 1.0)
    dn = (((1,), (1,)), ((), ()))
    acc = lax.dot_general(mean, wl_ref[...], dn,
                          preferred_element_type=jnp.float32)
    acc += lax.dot_general(f_ref[...], wr_ref[...], dn,
                           preferred_element_type=jnp.float32)
    acc += b_ref[...]
    if relu:
      acc = jnp.maximum(acc, 0.0)
    o_ref[...] = acc

  row_spec = pl.BlockSpec((blk, D), lambda i: (i, 0))
  full_spec = pl.BlockSpec((D, D), lambda i: (0, 0))
  b_spec = pl.BlockSpec((1, D), lambda i: (0, 0))
  return pl.pallas_call(
      body,
      grid=(grid,),
      in_specs=[row_spec, row_spec, row_spec, row_spec, row_spec,
                full_spec, b_spec, full_spec],
      out_specs=row_spec,
      out_shape=jax.ShapeDtypeStruct((n, D), jnp.float32),
  )(a0, a1, c0, c1, feat, w_l, b, w_r)


def kernel(x, edge_index, W1_l, b1, W1_r, W2_l, b2, W2_r):
  n = x.shape[0]
  e = edge_index.shape[1]
  n_pad = ((n // 256) + 1) * 256
  k_chunks = -(-e // (NT * CH * 8)) * 8
  e_pad = NT * CH * k_chunks

  src = edge_index[0]
  dst = edge_index[1]
  pad = e_pad - e
  pad_i = jnp.arange(pad, dtype=jnp.int32)
  src_p = jnp.concatenate([src, pad_i % n])
  dst_p = jnp.concatenate([dst, n + pad_i % (n_pad - n)])
  src2d = src_p.reshape(NT * k_chunks, CH)
  dst2d = dst_p.reshape(NT * k_chunks, CH)

  sc_agg_cnt = _make_sc_agg(n_pad, k_chunks, with_cnt=True)
  sc_agg = _make_sc_agg(n_pad, k_chunks, with_cnt=False)

  agg1, cnt = sc_agg_cnt(x, src2d, dst2d)
  b1r = b1.reshape(1, D)
  b2r = b2.reshape(1, D)
  h = _tc_fuse(agg1[0, :n], agg1[1, :n], cnt[0, :n], cnt[1, :n],
               x, W1_l, b1r, W1_r, relu=True)
  (agg2,) = sc_agg(h, src2d, dst2d)
  out = _tc_fuse(agg2[0, :n], agg2[1, :n], cnt[0, :n], cnt[1, :n],
                 h, W2_l, b2r, W2_r, relu=False)
  return out


# speedup vs baseline: 7.8926x; 1.0439x over previous
import functools

import jax
import jax.numpy as jnp
from jax import lax
from jax.experimental import pallas as pl
from jax.experimental.pallas import tpu as pltpu
from jax.experimental.pallas import tpu_sc as plsc

NC = 2
NS = 16
NT = NC * NS
CH = 80
D = 128


def _make_sc_agg(n_pad, k_chunks, with_cnt):
  mesh = plsc.VectorSubcoreMesh(core_axis_name="c", subcore_axis_name="s")
  rows_per_tile = n_pad // NS
  kb = 16

  out_type = [jax.ShapeDtypeStruct((NC, n_pad, D), jnp.float32)]
  if with_cnt:
    out_type.append(jax.ShapeDtypeStruct((NC, n_pad, D), jnp.float32))

  @functools.partial(
      pl.kernel,
      out_type=out_type,
      mesh=mesh,
      scratch_types=[
          pltpu.VMEM((kb, CH), jnp.int32),
          pltpu.VMEM((kb, CH), jnp.int32),
          pltpu.VMEM((CH, D), jnp.float32),
          pltpu.VMEM((CH, D), jnp.float32),
          pltpu.VMEM_SHARED((n_pad, D), jnp.float32),
          pltpu.SemaphoreType.DMA,
          pltpu.SemaphoreType.DMA,
      ])
  def body(feat_hbm, src_hbm, dst_hbm, agg_out, *rest):
    if with_cnt:
      cnt_out, src_v, dst_v, rows_a, rows_b, agg_sh, sem_g, sem_s = rest
    else:
      src_v, dst_v, rows_a, rows_b, agg_sh, sem_g, sem_s = rest
    cid = lax.axis_index("c")
    sid = lax.axis_index("s")
    wid = cid * NS + sid
    bufs = (rows_a, rows_b)
    zero16 = jnp.zeros((16,), jnp.float32)

    def fill_a(val):
      def fb(i, c):
        for j in range(D // 16):
          rows_a[i, pl.ds(j * 16, 16)] = zero16 + val
        return c
      lax.fori_loop(0, CH, fb, 0)

    def zero_acc():
      def zb(i, c):
        pltpu.sync_copy(rows_a.at[pl.ds(0, 64)],
                        agg_sh.at[pl.ds(sid * rows_per_tile + i * 64, 64)])
        return c
      lax.fori_loop(0, rows_per_tile // 64, zb, 0)

    def writeback(out_ref):
      def wb(i, c):
        base = sid * rows_per_tile + i * 64
        pltpu.sync_copy(agg_sh.at[pl.ds(base, 64)], rows_b.at[pl.ds(0, 64)])
        pltpu.sync_copy(rows_b.at[pl.ds(0, 64)],
                        out_ref.at[cid, pl.ds(base, 64)])
        return c
      lax.fori_loop(0, rows_per_tile // 64, wb, 0)

    if with_cnt:
      fill_a(0.0)
      zero_acc()
      fill_a(1.0)
      plsc.subcore_barrier()

      def cnt_block(b, c):
        base = wid * k_chunks + b * kb
        pltpu.sync_copy(dst_hbm.at[pl.ds(base, kb)], dst_v)
        for j in range(kb):
          pltpu.async_copy(rows_a, agg_sh.at[dst_v.at[j]], sem_s, add=True)
        for j in range(kb):
          pltpu.make_async_copy(rows_a, agg_sh.at[dst_v.at[0]],
                                sem_s).wait()
        return c
      lax.fori_loop(0, k_chunks // kb, cnt_block, 0)
      plsc.subcore_barrier()
      writeback(cnt_out)
      plsc.subcore_barrier()

    fill_a(0.0)
    zero_acc()
    plsc.subcore_barrier()

    def agg_block(b, c):
      base = wid * k_chunks + b * kb
      pltpu.sync_copy(src_hbm.at[pl.ds(base, kb)], src_v)
      pltpu.sync_copy(dst_hbm.at[pl.ds(base, kb)], dst_v)

      pltpu.async_copy(feat_hbm.at[src_v.at[0]], bufs[0], sem_g).wait()
      for j in range(kb):
        p = j % 2
        if j >= 1:
          pltpu.make_async_copy(bufs[1 - p], agg_sh.at[dst_v.at[j]],
                                sem_s).wait()
        if j + 1 < kb:
          g = pltpu.async_copy(feat_hbm.at[src_v.at[j + 1]], bufs[1 - p],
                               sem_g)
        pltpu.async_copy(bufs[p], agg_sh.at[dst_v.at[j]], sem_s, add=True)
        if j + 1 < kb:
          g.wait()
      pltpu.make_async_copy(bufs[(kb - 1) % 2], agg_sh.at[dst_v.at[0]],
                            sem_s).wait()
      return c
    lax.fori_loop(0, k_chunks // kb, agg_block, 0)
    plsc.subcore_barrier()
    writeback(agg_out)

  return body


def _tc_fuse(agg, cnt, feat, w_l, b, w_r, relu):
  n = feat.shape[0]
  blk = 400
  grid = n // blk

  def body(a0_ref, a1_ref, c0_ref, c1_ref, f_ref, wl_ref, b_ref, wr_ref,
           o_ref):
    cnt_col = c0_ref[0, :, 0:1] + c1_ref[0, :, 0:1]
    mean = (a0_ref[0] + a1_ref[0]) / jnp.maximum(cnt_col, 1.0)
    dn = (((1,), (1,)), ((), ()))
    acc = lax.dot_general(mean, wl_ref[...], dn,
                          preferred_element_type=jnp.float32)
    acc += lax.dot_general(f_ref[...], wr_ref[...], dn,
                           preferred_element_type=jnp.float32)
    acc += b_ref[...]
    if relu:
      acc = jnp.maximum(acc, 0.0)
    o_ref[...] = acc

  part0 = pl.BlockSpec((1, blk, D), lambda i: (0, i, 0))
  part1 = pl.BlockSpec((1, blk, D), lambda i: (1, i, 0))
  row_spec = pl.BlockSpec((blk, D), lambda i: (i, 0))
  full_spec = pl.BlockSpec((D, D), lambda i: (0, 0))
  b_spec = pl.BlockSpec((1, D), lambda i: (0, 0))
  return pl.pallas_call(
      body,
      grid=(grid,),
      in_specs=[part0, part1, part0, part1, row_spec,
                full_spec, b_spec, full_spec],
      out_specs=row_spec,
      out_shape=jax.ShapeDtypeStruct((n, D), jnp.float32),
  )(agg, agg, cnt, cnt, feat, w_l, b, w_r)


def kernel(x, edge_index, W1_l, b1, W1_r, W2_l, b2, W2_r):
  n = x.shape[0]
  e = edge_index.shape[1]
  n_pad = ((n // 256) + 1) * 256
  k_chunks = -(-e // (NT * CH * 8)) * 8
  e_pad = NT * CH * k_chunks

  src = edge_index[0]
  dst = edge_index[1]
  pad = e_pad - e
  pad_i = jnp.arange(pad, dtype=jnp.int32)
  src_p = jnp.concatenate([src, pad_i % n])
  dst_p = jnp.concatenate([dst, n + pad_i % (n_pad - n)])
  src2d = src_p.reshape(NT * k_chunks, CH)
  dst2d = dst_p.reshape(NT * k_chunks, CH)

  sc_agg_cnt = _make_sc_agg(n_pad, k_chunks, with_cnt=True)
  sc_agg = _make_sc_agg(n_pad, k_chunks, with_cnt=False)

  agg1, cnt = sc_agg_cnt(x, src2d, dst2d)
  b1r = b1.reshape(1, D)
  b2r = b2.reshape(1, D)
  h = _tc_fuse(agg1, cnt, x, W1_l, b1r, W1_r, relu=True)
  (agg2,) = sc_agg(h, src2d, dst2d)
  out = _tc_fuse(agg2, cnt, h, W2_l, b2r, W2_r, relu=False)
  return out


# speedup vs baseline: 9.6953x; 1.2284x over previous
import functools

import jax
import jax.numpy as jnp
from jax import lax
from jax.experimental import pallas as pl
from jax.experimental.pallas import tpu as pltpu
from jax.experimental.pallas import tpu_sc as plsc

NC = 2
NS = 16
NT = NC * NS
CH = 64
NBUF = 3
D = 128


def _make_sc_agg(n_pad, k_chunks, with_cnt):
  mesh = plsc.VectorSubcoreMesh(core_axis_name="c", subcore_axis_name="s")
  rows_per_tile = n_pad // NS
  kb = 16

  out_type = [jax.ShapeDtypeStruct((NC, n_pad, D), jnp.float32)]
  if with_cnt:
    out_type.append(jax.ShapeDtypeStruct((NC, n_pad, D), jnp.float32))

  @functools.partial(
      pl.kernel,
      out_type=out_type,
      mesh=mesh,
      scratch_types=[
          pltpu.VMEM((kb, CH), jnp.int32),
          pltpu.VMEM((kb, CH), jnp.int32),
      ] + [pltpu.VMEM((CH, D), jnp.float32) for _ in range(NBUF)] + [
          pltpu.VMEM_SHARED((n_pad, D), jnp.float32),
          pltpu.SemaphoreType.DMA,
          pltpu.SemaphoreType.DMA,
      ])
  def body(feat_hbm, src_hbm, dst_hbm, agg_out, *rest):
    if with_cnt:
      cnt_out, src_v, dst_v, *bufs, agg_sh, sem_g, sem_s = rest
    else:
      src_v, dst_v, *bufs, agg_sh, sem_g, sem_s = rest
    cid = lax.axis_index("c")
    sid = lax.axis_index("s")
    wid = cid * NS + sid
    rows_a, rows_b = bufs[0], bufs[1]
    zero16 = jnp.zeros((16,), jnp.float32)

    def fill_a(val):
      def fb(i, c):
        for j in range(D // 16):
          rows_a[i, pl.ds(j * 16, 16)] = zero16 + val
        return c
      lax.fori_loop(0, CH, fb, 0)

    def zero_acc():
      def zb(i, c):
        pltpu.sync_copy(rows_a.at[pl.ds(0, 64)],
                        agg_sh.at[pl.ds(sid * rows_per_tile + i * 64, 64)])
        return c
      lax.fori_loop(0, rows_per_tile // 64, zb, 0)

    def writeback(out_ref):
      def wb(i, c):
        base = sid * rows_per_tile + i * 64
        pltpu.sync_copy(agg_sh.at[pl.ds(base, 64)], rows_b.at[pl.ds(0, 64)])
        pltpu.sync_copy(rows_b.at[pl.ds(0, 64)],
                        out_ref.at[cid, pl.ds(base, 64)])
        return c
      lax.fori_loop(0, rows_per_tile // 64, wb, 0)

    if with_cnt:
      fill_a(0.0)
      zero_acc()
      fill_a(1.0)
      plsc.subcore_barrier()

      def cnt_block(b, c):
        base = wid * k_chunks + b * kb
        pltpu.sync_copy(dst_hbm.at[pl.ds(base, kb)], dst_v)
        for j in range(kb):
          pltpu.async_copy(rows_a, agg_sh.at[dst_v.at[j]], sem_s, add=True)
        for j in range(kb):
          pltpu.make_async_copy(rows_a, agg_sh.at[dst_v.at[0]],
                                sem_s).wait()
        return c
      lax.fori_loop(0, k_chunks // kb, cnt_block, 0)
      plsc.subcore_barrier()
      writeback(cnt_out)
      plsc.subcore_barrier()

    fill_a(0.0)
    zero_acc()
    plsc.subcore_barrier()

    def agg_block(b, c):
      base = wid * k_chunks + b * kb
      pltpu.sync_copy(src_hbm.at[pl.ds(base, kb)], src_v)
      pltpu.sync_copy(dst_hbm.at[pl.ds(base, kb)], dst_v)

      for j in range(NBUF - 1):
        pltpu.async_copy(feat_hbm.at[src_v.at[j]], bufs[j], sem_g)
      for j in range(kb):
        pltpu.make_async_copy(feat_hbm.at[src_v.at[0]], bufs[j % NBUF],
                              sem_g).wait()
        if j >= 1:
          pltpu.make_async_copy(bufs[0], agg_sh.at[dst_v.at[0]],
                                sem_s).wait()
        pltpu.async_copy(bufs[j % NBUF], agg_sh.at[dst_v.at[j]], sem_s,
                         add=True)
        nxt = j + NBUF - 1
        if nxt < kb:
          pltpu.async_copy(feat_hbm.at[src_v.at[nxt]], bufs[nxt % NBUF],
                           sem_g)
      pltpu.make_async_copy(bufs[0], agg_sh.at[dst_v.at[0]],
                            sem_s).wait()
      return c
    lax.fori_loop(0, k_chunks // kb, agg_block, 0)
    plsc.subcore_barrier()
    writeback(agg_out)

  return body


def _tc_fuse(agg, cnt, feat, w_l, b, w_r, relu):
  n = feat.shape[0]
  blk = 400
  grid = n // blk

  def body(a0_ref, a1_ref, c0_ref, c1_ref, f_ref, wl_ref, b_ref, wr_ref,
           o_ref):
    cnt_col = c0_ref[0, :, 0:1] + c1_ref[0, :, 0:1]
    mean = (a0_ref[0] + a1_ref[0]) / jnp.maximum(cnt_col, 1.0)
    dn = (((1,), (1,)), ((), ()))
    acc = lax.dot_general(mean, wl_ref[...], dn,
                          preferred_element_type=jnp.float32)
    acc += lax.dot_general(f_ref[...], wr_ref[...], dn,
                           preferred_element_type=jnp.float32)
    acc += b_ref[...]
    if relu:
      acc = jnp.maximum(acc, 0.0)
    o_ref[...] = acc

  part0 = pl.BlockSpec((1, blk, D), lambda i: (0, i, 0))
  part1 = pl.BlockSpec((1, blk, D), lambda i: (1, i, 0))
  row_spec = pl.BlockSpec((blk, D), lambda i: (i, 0))
  full_spec = pl.BlockSpec((D, D), lambda i: (0, 0))
  b_spec = pl.BlockSpec((1, D), lambda i: (0, 0))
  return pl.pallas_call(
      body,
      grid=(grid,),
      in_specs=[part0, part1, part0, part1, row_spec,
                full_spec, b_spec, full_spec],
      out_specs=row_spec,
      out_shape=jax.ShapeDtypeStruct((n, D), jnp.float32),
  )(agg, agg, cnt, cnt, feat, w_l, b, w_r)


def kernel(x, edge_index, W1_l, b1, W1_r, W2_l, b2, W2_r):
  n = x.shape[0]
  e = edge_index.shape[1]
  n_pad = ((n // 256) + 1) * 256
  k_chunks = -(-e // (NT * CH * 8)) * 8
  e_pad = NT * CH * k_chunks

  src = edge_index[0]
  dst = edge_index[1]
  pad = e_pad - e
  pad_i = jnp.arange(pad, dtype=jnp.int32)
  src_p = jnp.concatenate([src, pad_i % n])
  dst_p = jnp.concatenate([dst, n + pad_i % (n_pad - n)])
  src2d = src_p.reshape(NT * k_chunks, CH)
  dst2d = dst_p.reshape(NT * k_chunks, CH)

  sc_agg_cnt = _make_sc_agg(n_pad, k_chunks, with_cnt=True)
  sc_agg = _make_sc_agg(n_pad, k_chunks, with_cnt=False)

  agg1, cnt = sc_agg_cnt(x, src2d, dst2d)
  b1r = b1.reshape(1, D)
  b2r = b2.reshape(1, D)
  h = _tc_fuse(agg1, cnt, x, W1_l, b1r, W1_r, relu=True)
  (agg2,) = sc_agg(h, src2d, dst2d)
  out = _tc_fuse(agg2, cnt, h, W2_l, b2r, W2_r, relu=False)
  return out
